# baseline (speedup 1.0000x reference)
import jax
import jax.numpy as jnp
from jax.experimental import pallas as pl
from jax.experimental.pallas import tpu as pltpu

_M_SUB = 2
_U = 0.9
_INTERPRET = False
_BBLK = 128


def _band_mats(Wl, Wsp):
    O, C = Wl.shape[0], Wl.shape[1]
    E = jnp.stack([jnp.eye(Wsp, Wsp, k=2 - dx, dtype=Wl.dtype)
                   for dx in range(5)])
    M = jnp.einsum('ocdx,xvw->dcvow', Wl, E).reshape(5, C, Wsp, O, Wsp)
    Me = M[..., 0::2].reshape(5, C, Wsp, O * (Wsp // 2))
    Mo = M[..., 1::2].reshape(5, C, Wsp, O * (Wsp // 2))
    return Me, Mo


def _group_mat(n_in, n_out, group):
    ri = jax.lax.broadcasted_iota(jnp.int32, (n_in, n_out), 0)
    cj = jax.lax.broadcasted_iota(jnp.int32, (n_in, n_out), 1)
    return (cj // group == ri).astype(jnp.float32)


def _kernel_hash_bits(Wf, A):
    ss = jnp.sum(Wf * Wf, axis=1, keepdims=True)
    norms = jnp.sqrt(ss)
    scale = _U / (jnp.max(norms) + 1e-12)
    ws = Wf * scale
    n2 = jnp.sum(ws * ws, axis=1, keepdims=True)
    d = Wf.shape[1]
    dot = (jnp.dot(ws, A[:, :d].T, preferred_element_type=jnp.float32)
           + n2 * A[:, d][None, :] + (n2 * n2) * A[:, d + 1][None, :])
    return dot > 0


def _query_hash_bits(S_row, A, n_ch):
    d = n_ch * 25
    Ared = jnp.dot(A[:, :d], _group_mat(n_ch, d, 25).T,
                   preferred_element_type=jnp.float32)
    return jnp.dot(S_row, Ared.T, preferred_element_type=jnp.float32) > 0


def _mask_from_bits(kbits, qbits):
    match = jnp.all(kbits == qbits, axis=1, keepdims=True)
    mf = match.astype(jnp.float32)
    anyf = jnp.max(mf)
    return mf * anyf + (1.0 - anyf)


def _pool_hw(r2, Bblk, H, N, maskout=None):
    r = jnp.maximum(jnp.maximum(r2[:, :N], r2[:, N:]),
                    0.0).astype(jnp.bfloat16)
    if maskout is not None:
        r = r * maskout.astype(jnp.bfloat16)
    r4 = r.reshape(Bblk, H // 2, 2, N)
    return jnp.max(r4, axis=2)


def _make_mega(NB, Bblk):
    def _mega(x_ref, m1eo_ref, m2eo_ref, m3eo_ref,
              w1f_ref, a1_ref, w2f_ref, a2_ref, w3f_ref, a3_ref,
              wop_ref, bout_ref, out_ref,
              p1s_ref, p2s_ref, s1c_ref, s2c_ref, s3c_ref,
              xsum_ref, s1_ref, s2_ref,
              m1r_ref, m2r_ref, m3r_ref):
        i = pl.program_id(0)
        iloc = jax.lax.rem(i, NB)
        cdt = p1s_ref.dtype

        @pl.when(i == 0)
        def _init():
            xsum_ref[...] = jnp.zeros_like(xsum_ref)
            s1_ref[...] = jnp.zeros_like(s1_ref)
            s2_ref[...] = jnp.zeros_like(s2_ref)

        @pl.when(i < NB)
        def _phase_a():
            xs_parts = []
            for c in range(3):
                xc = x_ref[:, c]
                xcb = xc.astype(cdt)
                for dy in range(5):
                    s = dy - 2
                    a = max(0, -s)
                    b = 32 - max(0, s)
                    lo = (c * 5 + dy) * 32
                    if a > 0:
                        s1c_ref[:, 0:a, lo:lo + 32] = \
                            jnp.zeros((Bblk, a, 32), cdt)
                    if b < 32:
                        s1c_ref[:, b:32, lo:lo + 32] = \
                            jnp.zeros((Bblk, 32 - b, 32), cdt)
                    s1c_ref[:, a:b, lo:lo + 32] = xcb[:, a + s:b + s, :]
                xs_parts.append(jnp.sum(xc, axis=(0, 1))[None, :])
            xsum_ref[...] += jnp.concatenate(xs_parts, axis=1)
            sc = s1c_ref[...].reshape(Bblk * 32, 480)
            r2 = jnp.dot(sc, m1eo_ref[...],
                         preferred_element_type=jnp.float32)
            P = _pool_hw(r2, Bblk, 32, 256)
            p1s_ref[iloc, :, 2:18] = P
            z = jnp.zeros((Bblk, 2, 256), cdt)
            p1s_ref[iloc, :, 0:2] = z
            p1s_ref[iloc, :, 18:20] = z
            s1_ref[...] += jnp.sum(P.astype(jnp.float32),
                                   axis=(0, 1))[None, :]

        @pl.when(i == NB)
        def _sel12():
            S1c = jnp.dot(xsum_ref[...], _group_mat(3, 96, 32).T,
                          preferred_element_type=jnp.float32)
            kb1 = _kernel_hash_bits(w1f_ref[...], a1_ref[...])
            qb1 = _query_hash_bits(S1c, a1_ref[...], 3)
            m1 = _mask_from_bits(kb1, qb1)
            m1_row = m1.T
            m1r_ref[...] = jnp.dot(m1_row, _group_mat(16, 256, 16),
                                   preferred_element_type=jnp.float32)
            S2c = jnp.dot(s1_ref[...], _group_mat(16, 256, 16).T,
                          preferred_element_type=jnp.float32) * m1_row
            kb2 = _kernel_hash_bits(w2f_ref[...], a2_ref[...])
            qb2 = _query_hash_bits(S2c, a2_ref[...], 16)
            m2 = _mask_from_bits(kb2, qb2)
            m2r_ref[...] = jnp.dot(m2.T, _group_mat(20, 160, 8),
                                   preferred_element_type=jnp.float32)

        @pl.when((i >= NB) & (i < 2 * NB))
        def _phase_b():
            xb = p1s_ref[iloc]
            for dy in range(5):
                s2c_ref[:, :, dy * 256:(dy + 1) * 256] = xb[:, dy:dy + 16, :]
            m1cat = jnp.concatenate([m1r_ref[...]] * 5, axis=1).astype(cdt)
            sc = s2c_ref[...].reshape(Bblk * 16, 1280) * m1cat
            r2 = jnp.dot(sc, m2eo_ref[...],
                         preferred_element_type=jnp.float32)
            P = _pool_hw(r2, Bblk, 16, 160,
                         maskout=m2r_ref[...])
            p2s_ref[iloc, :, 2:10] = P
            z = jnp.zeros((Bblk, 2, 160), cdt)
            p2s_ref[iloc, :, 0:2] = z
            p2s_ref[iloc, :, 10:12] = z
            s2_ref[...] += jnp.sum(P.astype(jnp.float32),
                                   axis=(0, 1))[None, :]

        @pl.when(i == 2 * NB)
        def _sel3():
            S3c = jnp.dot(s2_ref[...], _group_mat(20, 160, 8).T,
                          preferred_element_type=jnp.float32)
            kb3 = _kernel_hash_bits(w3f_ref[...], a3_ref[...])
            qb3 = _query_hash_bits(S3c, a3_ref[...], 20)
            m3 = _mask_from_bits(kb3, qb3)
            m3r_ref[...] = jnp.dot(m3.T, _group_mat(20, 80, 4),
                                   preferred_element_type=jnp.float32)

        @pl.when(i >= 2 * NB)
        def _phase_c():
            xb = p2s_ref[iloc]
            for dy in range(5):
                s3c_ref[:, :, dy * 160:(dy + 1) * 160] = xb[:, dy:dy + 8, :]
            sc = s3c_ref[...].reshape(Bblk * 8, 800)
            r2 = jnp.dot(sc, m3eo_ref[...],
                         preferred_element_type=jnp.float32)
            P = _pool_hw(r2, Bblk, 8, 80,
                         maskout=m3r_ref[...])
            acc = jnp.zeros((Bblk, 10), jnp.float32)
            for hp in range(4):
                acc = acc + jnp.dot(P[:, hp, :].astype(jnp.float32),
                                    wop_ref[hp],
                                    preferred_element_type=jnp.float32)
            out_ref[...] = acc + bout_ref[...]

    return _mega


def kernel(x, W1, W2, W3, A1, A2, A3, Wout, bout):
    B = x.shape[0]
    Bblk = _BBLK
    NB = B // Bblk
    f32 = jnp.float32
    cdt = jnp.bfloat16
    x = x.astype(f32)

    M1e, M1o = _band_mats(W1.astype(cdt), 32)
    M1eo = jnp.concatenate(
        [jnp.transpose(M1e, (1, 0, 2, 3)).reshape(480, 256),
         jnp.transpose(M1o, (1, 0, 2, 3)).reshape(480, 256)], axis=1)
    M2e, M2o = _band_mats(W2.astype(cdt), 16)
    M2eo = jnp.concatenate(
        [M2e.reshape(1280, 160), M2o.reshape(1280, 160)], axis=1)
    M3e, M3o = _band_mats(W3.astype(cdt), 8)
    M3eo = jnp.concatenate(
        [M3e.reshape(800, 80), M3o.reshape(800, 80)], axis=1)
    W1f = W1.reshape(16, 75).astype(f32)
    W2f = W2.reshape(20, 400).astype(f32)
    W3f = W3.reshape(20, 500).astype(f32)
    WoP = jnp.transpose(Wout.reshape(10, 20, 4, 4), (2, 1, 3, 0)) \
             .reshape(4, 80, 10).astype(f32)

    const2 = lambda i: (0, 0)
    const3 = lambda i: (0, 0, 0)

    out = pl.pallas_call(
        _make_mega(NB, Bblk),
        grid=(3 * NB,),
        in_specs=[
            pl.BlockSpec((Bblk, 3, 32, 32),
                         lambda i: (jnp.minimum(i, NB - 1), 0, 0, 0)),
            pl.BlockSpec((480, 512), const2),
            pl.BlockSpec((1280, 320), const2),
            pl.BlockSpec((800, 160), const2),
            pl.BlockSpec((16, 75), const2),
            pl.BlockSpec((2, 77), const2),
            pl.BlockSpec((20, 400), const2),
            pl.BlockSpec((2, 402), const2),
            pl.BlockSpec((20, 500), const2),
            pl.BlockSpec((3, 502), const2),
            pl.BlockSpec((4, 80, 10), const3),
            pl.BlockSpec((1, 10), const2),
        ],
        out_specs=pl.BlockSpec((Bblk, 10),
                               lambda i: (jnp.maximum(i - 2 * NB, 0), 0)),
        out_shape=jax.ShapeDtypeStruct((B, 10), f32),
        scratch_shapes=[
            pltpu.VMEM((NB, Bblk, 20, 256), cdt),
            pltpu.VMEM((NB, Bblk, 12, 160), cdt),
            pltpu.VMEM((Bblk, 32, 480), cdt),
            pltpu.VMEM((Bblk, 16, 1280), cdt),
            pltpu.VMEM((Bblk, 8, 800), cdt),
            pltpu.VMEM((1, 96), f32),
            pltpu.VMEM((1, 256), f32),
            pltpu.VMEM((1, 160), f32),
            pltpu.VMEM((1, 256), f32),
            pltpu.VMEM((1, 160), f32),
            pltpu.VMEM((1, 80), f32),
        ],
        compiler_params=pltpu.CompilerParams(
            dimension_semantics=("arbitrary",)),
        interpret=_INTERPRET,
    )(x, M1eo, M2eo, M3eo,
      W1f, A1.astype(f32), W2f, A2.astype(f32), W3f, A3.astype(f32),
      WoP, bout.reshape(1, 10).astype(f32))

    return out


# speedup vs baseline: 3.3158x; 3.3158x over previous
import jax
import jax.numpy as jnp
from jax.experimental import pallas as pl
from jax.experimental.pallas import tpu as pltpu

_M_SUB = 2
_U = 0.9
_INTERPRET = False
_BBLK = 128


def _band_mats(Wl, Wsp):
    O, C = Wl.shape[0], Wl.shape[1]
    E = jnp.stack([jnp.eye(Wsp, Wsp, k=2 - dx, dtype=Wl.dtype)
                   for dx in range(5)])
    M = jnp.einsum('ocdx,xvw->dcvow', Wl, E).reshape(5, C * Wsp, O, Wsp)
    Me = M[..., 0::2].reshape(5, C * Wsp, O * (Wsp // 2))
    Mo = M[..., 1::2].reshape(5, C * Wsp, O * (Wsp // 2))
    return Me, Mo


def _conv_pool(sc, me, mo, Hout, Bblk, maskout=None):
    N = me.shape[1]
    cdt = sc.dtype
    re = jnp.dot(sc, me, preferred_element_type=jnp.float32)
    ro = jnp.dot(sc, mo, preferred_element_type=jnp.float32)
    r = jnp.maximum(jnp.maximum(re, ro), 0.0).astype(cdt)
    if maskout is not None:
        r = r * maskout.astype(cdt)
    r = r.reshape(Hout // 2, 2, Bblk, N)
    return jnp.max(r, axis=1)


def _conv_pool_eo(sc, meo, Hout, Bblk, N, maskout=None):
    cdt = sc.dtype
    r2 = jnp.dot(sc, meo, preferred_element_type=jnp.float32)
    r = jnp.maximum(jnp.maximum(r2[:, :N], r2[:, N:]), 0.0).astype(cdt)
    if maskout is not None:
        r = r * maskout.astype(cdt)
    r = r.reshape(Hout // 2, 2, Bblk, N)
    return jnp.max(r, axis=1)


def _group_mat(n_in, n_out, group):
    ri = jax.lax.broadcasted_iota(jnp.int32, (n_in, n_out), 0)
    cj = jax.lax.broadcasted_iota(jnp.int32, (n_in, n_out), 1)
    return (cj // group == ri).astype(jnp.float32)


def _kernel_hash_bits(Wf, A):
    ss = jnp.sum(Wf * Wf, axis=1, keepdims=True)
    norms = jnp.sqrt(ss)
    scale = _U / (jnp.max(norms) + 1e-12)
    ws = Wf * scale
    n2 = jnp.sum(ws * ws, axis=1, keepdims=True)
    d = Wf.shape[1]
    dot = (jnp.dot(ws, A[:, :d].T, preferred_element_type=jnp.float32)
           + n2 * A[:, d][None, :] + (n2 * n2) * A[:, d + 1][None, :])
    return dot > 0


def _query_hash_bits(S_row, A, n_ch):
    d = n_ch * 25
    Ared = jnp.dot(A[:, :d], _group_mat(n_ch, d, 25).T,
                   preferred_element_type=jnp.float32)
    return jnp.dot(S_row, Ared.T, preferred_element_type=jnp.float32) > 0


def _mask_from_bits(kbits, qbits):
    match = jnp.all(kbits == qbits, axis=1, keepdims=True)
    mf = match.astype(jnp.float32)
    anyf = jnp.max(mf)
    return mf * anyf + (1.0 - anyf)


def _make_mega(NB, Bblk):
    def _mega(xp_ref, m1e_ref, m1o_ref, m2eo_ref, m3eo_ref,
              w1f_ref, a1_ref, w2f_ref, a2_ref, w3f_ref, a3_ref,
              wop_ref, bout_ref, out_ref,
              p1s_ref, p2s_ref, s1c_ref, s2c_ref, s3c_ref,
              xsum_ref, s1_ref, s2_ref,
              m1r_ref, m2r_ref, m3r_ref):
        i = pl.program_id(0)
        iloc = jax.lax.rem(i, NB)
        cdt = p1s_ref.dtype

        @pl.when(i == 0)
        def _init():
            xsum_ref[...] = jnp.zeros_like(xsum_ref)
            s1_ref[...] = jnp.zeros_like(s1_ref)
            s2_ref[...] = jnp.zeros_like(s2_ref)
            s1c_ref[...] = jnp.zeros_like(s1c_ref)
            s3c_ref[...] = jnp.zeros_like(s3c_ref)

        @pl.when(i < NB)
        def _phase_a():
            for dy in range(5):
                s = dy - 2
                a = max(0, -s)
                b = 32 - max(0, s)
                s1c_ref[a * Bblk:b * Bblk, dy * 96:dy * 96 + 96] = \
                    xp_ref[a + s:b + s].reshape((b - a) * Bblk,
                                                96).astype(cdt)
            P = _conv_pool(s1c_ref[...], m1e_ref[...], m1o_ref[...],
                           32, Bblk)
            p1s_ref[iloc, 2:18] = P.astype(cdt)
            z = jnp.zeros((2, Bblk, 256), cdt)
            p1s_ref[iloc, 0:2] = z
            p1s_ref[iloc, 18:20] = z
            xsum_ref[...] += jnp.sum(xp_ref[...], axis=(0, 1))[None, :]
            s1_ref[...] += jnp.sum(P.astype(jnp.float32), axis=(0, 1))[None, :]

        @pl.when(i == NB)
        def _sel12():
            S1c = jnp.dot(xsum_ref[...], _group_mat(3, 96, 32).T,
                          preferred_element_type=jnp.float32)
            kb1 = _kernel_hash_bits(w1f_ref[...], a1_ref[...])
            qb1 = _query_hash_bits(S1c, a1_ref[...], 3)
            m1 = _mask_from_bits(kb1, qb1)
            m1_row = m1.T
            m1r_ref[...] = jnp.dot(m1_row, _group_mat(16, 256, 16),
                                   preferred_element_type=jnp.float32)
            S2c = jnp.dot(s1_ref[...], _group_mat(16, 256, 16).T,
                          preferred_element_type=jnp.float32) * m1_row
            kb2 = _kernel_hash_bits(w2f_ref[...], a2_ref[...])
            qb2 = _query_hash_bits(S2c, a2_ref[...], 16)
            m2 = _mask_from_bits(kb2, qb2)
            m2r_ref[...] = jnp.dot(m2.T, _group_mat(20, 160, 8),
                                   preferred_element_type=jnp.float32)

        @pl.when((i >= NB) & (i < 2 * NB))
        def _phase_b():
            xb = p1s_ref[iloc]
            for dy in range(5):
                s2c_ref[:, dy * 256:(dy + 1) * 256] = \
                    xb[dy:dy + 16].reshape(16 * Bblk, 256)
            m1cat = jnp.concatenate([m1r_ref[...]] * 5, axis=1).astype(cdt)
            sc = s2c_ref[...] * m1cat
            P = _conv_pool_eo(sc, m2eo_ref[...], 16, Bblk, 160,
                              maskout=m2r_ref[...])
            p2s_ref[iloc, 2:10] = P.astype(cdt)
            z = jnp.zeros((2, Bblk, 160), cdt)
            p2s_ref[iloc, 0:2] = z
            p2s_ref[iloc, 10:12] = z
            s2_ref[...] += jnp.sum(P.astype(jnp.float32), axis=(0, 1))[None, :]

        @pl.when(i == 2 * NB)
        def _sel3():
            S3c = jnp.dot(s2_ref[...], _group_mat(20, 160, 8).T,
                          preferred_element_type=jnp.float32)
            kb3 = _kernel_hash_bits(w3f_ref[...], a3_ref[...])
            qb3 = _query_hash_bits(S3c, a3_ref[...], 20)
            m3 = _mask_from_bits(kb3, qb3)
            m3r_ref[...] = jnp.dot(m3.T, _group_mat(20, 80, 4),
                                   preferred_element_type=jnp.float32)

        @pl.when(i >= 2 * NB)
        def _phase_c():
            x3 = p2s_ref[iloc]
            for dy in range(5):
                s3c_ref[:, dy * 160:dy * 160 + 160] = \
                    x3[dy:dy + 8].reshape(8 * Bblk, 160)
            P = _conv_pool_eo(s3c_ref[...], m3eo_ref[...], 8, Bblk, 80,
                              maskout=m3r_ref[...])
            acc = jnp.zeros((Bblk, 10), jnp.float32)
            for hp in range(4):
                acc = acc + jnp.dot(P[hp], wop_ref[hp],
                                    preferred_element_type=jnp.float32)
            out_ref[...] = acc + bout_ref[...]

    return _mega


def kernel(x, W1, W2, W3, A1, A2, A3, Wout, bout):
    B = x.shape[0]
    Bblk = _BBLK
    NB = B // Bblk
    f32 = jnp.float32
    cdt = jnp.bfloat16

    xp = jnp.transpose(x, (2, 0, 1, 3)).reshape(32, B, 96)
    M1e, M1o = _band_mats(W1.astype(cdt), 32)
    M1e = jnp.pad(M1e.reshape(480, 256), ((0, 32), (0, 0)))
    M1o = jnp.pad(M1o.reshape(480, 256), ((0, 32), (0, 0)))
    M2e, M2o = _band_mats(W2.astype(cdt), 16)
    M2eo = jnp.concatenate(
        [M2e.reshape(1280, 160), M2o.reshape(1280, 160)], axis=1)
    M3e, M3o = _band_mats(W3.astype(cdt), 8)
    M3eo = jnp.pad(jnp.concatenate(
        [M3e.reshape(800, 80), M3o.reshape(800, 80)], axis=1),
        ((0, 96), (0, 0)))
    W1f = W1.reshape(16, 75).astype(f32)
    W2f = W2.reshape(20, 400).astype(f32)
    W3f = W3.reshape(20, 500).astype(f32)
    WoP = jnp.transpose(Wout.reshape(10, 20, 4, 4), (2, 1, 3, 0)) \
             .reshape(4, 80, 10).astype(f32)

    const2 = lambda i: (0, 0)
    const3 = lambda i: (0, 0, 0)

    out = pl.pallas_call(
        _make_mega(NB, Bblk),
        grid=(3 * NB,),
        in_specs=[
            pl.BlockSpec((32, Bblk, 96),
                         lambda i: (0, jnp.minimum(i, NB - 1), 0)),
            pl.BlockSpec((512, 256), const2),
            pl.BlockSpec((512, 256), const2),
            pl.BlockSpec((1280, 320), const2),
            pl.BlockSpec((896, 160), const2),
            pl.BlockSpec((16, 75), const2),
            pl.BlockSpec((2, 77), const2),
            pl.BlockSpec((20, 400), const2),
            pl.BlockSpec((2, 402), const2),
            pl.BlockSpec((20, 500), const2),
            pl.BlockSpec((3, 502), const2),
            pl.BlockSpec((4, 80, 10), const3),
            pl.BlockSpec((1, 10), const2),
        ],
        out_specs=pl.BlockSpec((Bblk, 10),
                               lambda i: (jnp.maximum(i - 2 * NB, 0), 0)),
        out_shape=jax.ShapeDtypeStruct((B, 10), f32),
        scratch_shapes=[
            pltpu.VMEM((NB, 20, Bblk, 256), cdt),
            pltpu.VMEM((NB, 12, Bblk, 160), cdt),
            pltpu.VMEM((32 * Bblk, 512), cdt),
            pltpu.VMEM((16 * Bblk, 1280), cdt),
            pltpu.VMEM((8 * Bblk, 896), cdt),
            pltpu.VMEM((1, 96), f32),
            pltpu.VMEM((1, 256), f32),
            pltpu.VMEM((1, 160), f32),
            pltpu.VMEM((1, 256), f32),
            pltpu.VMEM((1, 160), f32),
            pltpu.VMEM((1, 80), f32),
        ],
        compiler_params=pltpu.CompilerParams(
            dimension_semantics=("arbitrary",)),
        interpret=_INTERPRET,
    )(xp, M1e, M1o, M2eo, M3eo,
      W1f, A1.astype(f32), W2f, A2.astype(f32), W3f, A3.astype(f32),
      WoP, bout.reshape(1, 10).astype(f32))

    return out
